# baseline (speedup 1.0000x reference)
import jax
import jax.numpy as jnp
from jax.experimental import pallas as pl

_ROWS = 16384
_COLS = 2048
_BLK = 512
_DST_ROW = 4352
_DST_COL = 1
_SRC_BLK_ROW = 800


def _ew_kernel(x_ref, src_ref, o_ref):
    i = pl.program_id(0)
    y = 2.0 * jnp.log(x_ref[...])
    dst_blk = _DST_ROW // _BLK

    @pl.when(i == dst_blk)
    def _():
        s = 2.0 * jnp.log(src_ref[0, 128])
        rows = jax.lax.broadcasted_iota(jnp.int32, (_BLK, _COLS), 0)
        cols = jax.lax.broadcasted_iota(jnp.int32, (_BLK, _COLS), 1)
        mask = (rows == (_DST_ROW - dst_blk * _BLK)) & (cols == _DST_COL)
        o_ref[...] = jnp.where(mask, s, y)

    @pl.when(i != dst_blk)
    def _():
        o_ref[...] = y


def kernel(x):
    xr = x.reshape(_ROWS, _COLS)
    out = pl.pallas_call(
        _ew_kernel,
        grid=(_ROWS // _BLK,),
        in_specs=[
            pl.BlockSpec((_BLK, _COLS), lambda i: (i, 0)),
            pl.BlockSpec((8, _COLS), lambda i: (_SRC_BLK_ROW, 0)),
        ],
        out_specs=pl.BlockSpec((_BLK, _COLS), lambda i: (i, 0)),
        out_shape=jax.ShapeDtypeStruct((_ROWS, _COLS), x.dtype),
    )(xr, xr)
    return out.reshape(x.shape)


# speedup vs baseline: 1.0088x; 1.0088x over previous
import jax
import jax.numpy as jnp
from jax.experimental import pallas as pl

_ROWS = 16384
_COLS = 2048
_BLK = 512
_DST_ROW = 4352
_DST_COL = 1
_SRC_BLK_ROW = 800


def _ew_kernel(x_ref, src_ref, o_ref):
    i = pl.program_id(0)
    dst_blk = _DST_ROW // _BLK

    @pl.when(i == dst_blk)
    def _():
        s = 2.0 * jnp.log(src_ref[0, 128])
        rows = jax.lax.broadcasted_iota(jnp.int32, (_BLK, _COLS), 0)
        cols = jax.lax.broadcasted_iota(jnp.int32, (_BLK, _COLS), 1)
        mask = (rows == (_DST_ROW - dst_blk * _BLK)) & (cols == _DST_COL)
        o_ref[...] = jnp.where(mask, s, 2.0 * jnp.log(x_ref[...]))

    @pl.when(i != dst_blk)
    def _():
        o_ref[...] = 2.0 * jnp.log(x_ref[...])


def kernel(x):
    xr = x.reshape(_ROWS, _COLS)
    out = pl.pallas_call(
        _ew_kernel,
        grid=(_ROWS // _BLK,),
        in_specs=[
            pl.BlockSpec((_BLK, _COLS), lambda i: (i, 0)),
            pl.BlockSpec((8, _COLS), lambda i: (_SRC_BLK_ROW, 0)),
        ],
        out_specs=pl.BlockSpec((_BLK, _COLS), lambda i: (i, 0)),
        out_shape=jax.ShapeDtypeStruct((_ROWS, _COLS), x.dtype),
    )(xr, xr)
    return out.reshape(x.shape)


# speedup vs baseline: 4.2598x; 4.2227x over previous
import jax
import jax.numpy as jnp
from jax.experimental import pallas as pl

_LEAD = 128
_R = 2048
_C = 128
_BLK = 4
_DST = (34, 0, 1)
_SRC = (50, 1, 0)


def _ew_kernel(x_ref, src_ref, o_ref):
    i = pl.program_id(0)
    dst_blk = _DST[0] // _BLK

    @pl.when(i == dst_blk)
    def _():
        s = 2.0 * jnp.log(src_ref[0, _SRC[1], _SRC[2]])
        d0 = jax.lax.broadcasted_iota(jnp.int32, (_BLK, _R, _C), 0)
        d1 = jax.lax.broadcasted_iota(jnp.int32, (_BLK, _R, _C), 1)
        d2 = jax.lax.broadcasted_iota(jnp.int32, (_BLK, _R, _C), 2)
        mask = (
            (d0 == _DST[0] - dst_blk * _BLK) & (d1 == _DST[1]) & (d2 == _DST[2])
        )
        o_ref[...] = jnp.where(mask, s, 2.0 * jnp.log(x_ref[...]))

    @pl.when(i != dst_blk)
    def _():
        o_ref[...] = 2.0 * jnp.log(x_ref[...])


def kernel(x):
    xr = x.reshape(_LEAD, _R, _C)
    out = pl.pallas_call(
        _ew_kernel,
        grid=(_LEAD // _BLK,),
        in_specs=[
            pl.BlockSpec((_BLK, _R, _C), lambda i: (i, 0, 0)),
            pl.BlockSpec((1, 8, _C), lambda i: (_SRC[0], 0, 0)),
        ],
        out_specs=pl.BlockSpec((_BLK, _R, _C), lambda i: (i, 0, 0)),
        out_shape=jax.ShapeDtypeStruct((_LEAD, _R, _C), x.dtype),
    )(xr, xr)
    return out.reshape(x.shape)


# speedup vs baseline: 4.3683x; 1.0255x over previous
import jax
import jax.numpy as jnp
from jax.experimental import pallas as pl

_LEAD = 128
_R = 2048
_C = 128
_BLK = 8
_DST = (34, 0, 1)
_SRC = (50, 1, 0)


def _ew_kernel(x_ref, src_ref, o_ref):
    i = pl.program_id(0)
    dst_blk = _DST[0] // _BLK

    @pl.when(i == dst_blk)
    def _():
        s = 2.0 * jnp.log(src_ref[0, _SRC[1], _SRC[2]])
        d0 = jax.lax.broadcasted_iota(jnp.int32, (_BLK, _R, _C), 0)
        d1 = jax.lax.broadcasted_iota(jnp.int32, (_BLK, _R, _C), 1)
        d2 = jax.lax.broadcasted_iota(jnp.int32, (_BLK, _R, _C), 2)
        mask = (
            (d0 == _DST[0] - dst_blk * _BLK) & (d1 == _DST[1]) & (d2 == _DST[2])
        )
        o_ref[...] = jnp.where(mask, s, 2.0 * jnp.log(x_ref[...]))

    @pl.when(i != dst_blk)
    def _():
        o_ref[...] = 2.0 * jnp.log(x_ref[...])


def kernel(x):
    xr = x.reshape(_LEAD, _R, _C)
    out = pl.pallas_call(
        _ew_kernel,
        grid=(_LEAD // _BLK,),
        in_specs=[
            pl.BlockSpec((_BLK, _R, _C), lambda i: (i, 0, 0)),
            pl.BlockSpec((1, 8, _C), lambda i: (_SRC[0], 0, 0)),
        ],
        out_specs=pl.BlockSpec((_BLK, _R, _C), lambda i: (i, 0, 0)),
        out_shape=jax.ShapeDtypeStruct((_LEAD, _R, _C), x.dtype),
    )(xr, xr)
    return out.reshape(x.shape)
